# baseline (speedup 1.0000x reference)
import jax
import jax.numpy as jnp
from jax import lax
from jax.experimental import pallas as pl
from jax.experimental.pallas import tpu as pltpu
from jax.experimental.pallas import tpu_sc as plsc

NC = 2
NS = 16
NW = NC * NS
B = 16384
D = 32
PACK = 128 // D
BPW = B // NW
CHUNK = 128
NCHUNK = BPW // CHUNK


NBLK_FULL = 7812
BLK_BASE = NBLK_FULL // NW
BLK_EXTRA = NBLK_FULL % NW
TAIL_R0 = NBLK_FULL * 32


def _transpose_body(uwt_hbm, iwt_hbm, utail_hbm, itail_hbm, ou_hbm, oi_hbm,
                    in_u, in_i, tb_u, tb_i, tail_v):
    wid = lax.axis_index("s") * NC + lax.axis_index("c")
    start = wid * BLK_BASE + jnp.minimum(wid, BLK_EXTRA)
    count = BLK_BASE + (wid < BLK_EXTRA).astype(jnp.int32)
    lane = jnp.arange(16, dtype=jnp.int32)
    rows_o = []
    colb_o = []
    for o in range(8):
        l = o * 16 + lane
        rows_o.append(l >> 2)
        colb_o.append((l & 3) * D)

    def body(b, carry):
        blk = start + b
        u0 = pl.multiple_of(blk * 128, 128)
        pltpu.sync_copy(uwt_hbm.at[:, pl.ds(u0, 128)], in_u)
        pltpu.sync_copy(iwt_hbm.at[:, pl.ds(u0, 128)], in_i)
        for d in range(D):
            for o in range(8):
                plsc.store_scatter(
                    tb_u, [rows_o[o], colb_o[o] + d], in_u[d, pl.ds(o * 16, 16)])
                plsc.store_scatter(
                    tb_i, [rows_o[o], colb_o[o] + d], in_i[d, pl.ds(o * 16, 16)])
        r0 = pl.multiple_of(blk * 32, 32)
        pltpu.sync_copy(tb_u, ou_hbm.at[pl.ds(r0, 32)])
        pltpu.sync_copy(tb_i, oi_hbm.at[pl.ds(r0, 32)])
        return carry

    lax.fori_loop(0, count, body, 0)

    @pl.when(wid == 0)
    def _():
        pltpu.sync_copy(utail_hbm, tail_v)
        pltpu.sync_copy(tail_v, ou_hbm.at[pl.ds(TAIL_R0, 16)])
        pltpu.sync_copy(itail_hbm, tail_v)
        pltpu.sync_copy(tail_v, oi_hbm.at[pl.ds(TAIL_R0, 16)])


def _gmf_body(user_hbm, item_hbm, uw_hbm, iw_hbm, params_hbm, out_hbm,
              uidx_v, iidx_v, udma_v, idma_v, ucol_v, icol_v,
              ub0, ub1, ib0, ib1, params_v, out_v, sem0, sem1):
    wid = lax.axis_index("s") * NC + lax.axis_index("c")
    pltpu.sync_copy(user_hbm.at[wid], uidx_v)
    pltpu.sync_copy(item_hbm.at[wid], iidx_v)
    pltpu.sync_copy(params_hbm, params_v)

    for j in range(NCHUNK):
        for k in range(CHUNK // 16):
            uv = uidx_v[j, pl.ds(k * 16, 16)]
            iv = iidx_v[j, pl.ds(k * 16, 16)]
            udma_v[j, pl.ds(k * 16, 16)] = uv >> 2
            idma_v[j, pl.ds(k * 16, 16)] = iv >> 2
            ucol_v[pl.ds(j * CHUNK + k * 16, 16)] = (uv & 3) * D
            icol_v[pl.ds(j * CHUNK + k * 16, 16)] = (iv & 3) * D

    w_lo = params_v[pl.ds(0, 16)]
    w_hi = params_v[pl.ds(16, 16)]
    bias = params_v[pl.ds(32, 16)][0]
    wd = [w_lo[d] for d in range(16)] + [w_hi[d - 16] for d in range(16, D)]
    lane = jnp.arange(16, dtype=jnp.int32)

    ubufs = [ub0, ub1]
    ibufs = [ib0, ib1]
    sems = [sem0, sem1]

    def fire(c):
        s = sems[c % 2]
        return (pltpu.async_copy(uw_hbm.at[udma_v.at[c]], ubufs[c % 2], s),
                pltpu.async_copy(iw_hbm.at[idma_v.at[c]], ibufs[c % 2], s))

    pending = fire(0)
    for c in range(NCHUNK):
        nxt = fire(c + 1) if c + 1 < NCHUNK else None
        for p in pending:
            p.wait()
        ubuf = ubufs[c % 2]
        ibuf = ibufs[c % 2]

        def body(g, carry):
            rows = g * 16 + lane
            ucol = ucol_v[pl.ds(c * CHUNK + g * 16, 16)]
            icol = icol_v[pl.ds(c * CHUNK + g * 16, 16)]
            acc = jnp.zeros((16,), jnp.float32)
            for d in range(D):
                u = plsc.load_gather(ubuf, [rows, ucol + d])
                i = plsc.load_gather(ibuf, [rows, icol + d])
                acc = acc + (u * i) * wd[d]
            out_v[pl.ds(c * CHUNK + g * 16, 16)] = acc + bias
            return carry

        lax.fori_loop(0, CHUNK // 16, body, 0)
        pending = nxt

    pltpu.sync_copy(out_v, out_hbm.at[wid])


def kernel(user, item, embed_user_weight, embed_item_weight, predict_weight,
           predict_bias):
    uw_t = embed_user_weight.T
    iw_t = embed_item_weight.T
    utail = embed_user_weight[NBLK_FULL * 128:].reshape(16, 128)
    itail = embed_item_weight[NBLK_FULL * 128:].reshape(16, 128)
    mesh = plsc.VectorSubcoreMesh(core_axis_name="c", subcore_axis_name="s")
    tr = pl.kernel(
        _transpose_body,
        out_type=(jax.ShapeDtypeStruct((250000, 128), jnp.float32),
                  jax.ShapeDtypeStruct((250000, 128), jnp.float32)),
        mesh=mesh,
        scratch_types=[
            pltpu.VMEM((D, 128), jnp.float32),
            pltpu.VMEM((D, 128), jnp.float32),
            pltpu.VMEM((32, 128), jnp.float32),
            pltpu.VMEM((32, 128), jnp.float32),
            pltpu.VMEM((16, 128), jnp.float32),
        ],
        compiler_params=pltpu.CompilerParams(
            needs_layout_passes=False, use_tc_tiling_on_sc=True),
    )
    uw_p, iw_p = tr(uw_t, iw_t, utail, itail)
    user3 = user.reshape(NW, NCHUNK, CHUNK)
    item3 = item.reshape(NW, NCHUNK, CHUNK)
    params = jnp.concatenate([
        predict_weight.reshape(D), predict_bias,
        jnp.zeros((15,), jnp.float32)])
    k = pl.kernel(
        _gmf_body,
        out_type=jax.ShapeDtypeStruct((NW, BPW), jnp.float32),
        mesh=mesh,
        scratch_types=[
            pltpu.VMEM((NCHUNK, CHUNK), jnp.int32),
            pltpu.VMEM((NCHUNK, CHUNK), jnp.int32),
            pltpu.VMEM((NCHUNK, CHUNK), jnp.int32),
            pltpu.VMEM((NCHUNK, CHUNK), jnp.int32),
            pltpu.VMEM((BPW,), jnp.int32),
            pltpu.VMEM((BPW,), jnp.int32),
            pltpu.VMEM((CHUNK, 128), jnp.float32),
            pltpu.VMEM((CHUNK, 128), jnp.float32),
            pltpu.VMEM((CHUNK, 128), jnp.float32),
            pltpu.VMEM((CHUNK, 128), jnp.float32),
            pltpu.VMEM((48,), jnp.float32),
            pltpu.VMEM((BPW,), jnp.float32),
            pltpu.SemaphoreType.DMA,
            pltpu.SemaphoreType.DMA,
        ],
        compiler_params=pltpu.CompilerParams(
            needs_layout_passes=False, use_tc_tiling_on_sc=True),
    )
    out = k(user3, item3, uw_p, iw_p, params)
    return out.reshape(B)


# speedup vs baseline: 1.2889x; 1.2889x over previous
import jax
import jax.numpy as jnp
from jax import lax
from jax.experimental import pallas as pl
from jax.experimental.pallas import tpu as pltpu
from jax.experimental.pallas import tpu_sc as plsc

NC = 2
NS = 16
NW = NC * NS
B = 16384
D = 32
PACK = 128 // D
BPW = B // NW
CHUNK = 128
NCHUNK = BPW // CHUNK


NBLK_FULL = 7812
BLK_BASE = NBLK_FULL // NW
BLK_EXTRA = NBLK_FULL % NW
TAIL_R0 = NBLK_FULL * 32


def _transpose_body(uwt_hbm, iwt_hbm, utail_hbm, itail_hbm, ou_hbm, oi_hbm,
                    inu0, ini0, inu1, ini1, tb_u, tb_i, tail_v, semA, semB):
    wid = lax.axis_index("s") * NC + lax.axis_index("c")
    start = wid * BLK_BASE + jnp.minimum(wid, BLK_EXTRA)
    count = BLK_BASE + (wid < BLK_EXTRA).astype(jnp.int32)
    lane = jnp.arange(16, dtype=jnp.int32)
    base_o = []
    for o in range(8):
        l = o * 16 + lane
        base_o.append((l >> 2) * 128 + (l & 3) * D)

    ins = [(inu0, ini0, semA), (inu1, ini1, semB)]

    def fire(setid, blk):
        inu, ini, sem = ins[setid]
        u0 = pl.multiple_of(blk * 128, 128)
        pltpu.async_copy(uwt_hbm.at[:, pl.ds(u0, 128)], inu, sem)
        pltpu.async_copy(iwt_hbm.at[:, pl.ds(u0, 128)], ini, sem)

    def drain(setid):
        inu, ini, sem = ins[setid]
        pltpu.make_async_copy(uwt_hbm.at[:, pl.ds(0, 128)], inu, sem).wait()
        pltpu.make_async_copy(iwt_hbm.at[:, pl.ds(0, 128)], ini, sem).wait()

    def process(b, setid):
        blk = start + b
        drain(setid)

        @pl.when(b + 1 < count)
        def _():
            fire(1 - setid, blk + 1)

        inu, ini, _ = ins[setid]
        for d in range(D):
            for o in range(8):
                plsc.store_scatter(tb_u, [base_o[o] + d],
                                   inu[d, pl.ds(o * 16, 16)])
                plsc.store_scatter(tb_i, [base_o[o] + d],
                                   ini[d, pl.ds(o * 16, 16)])
        e0 = pl.multiple_of(blk * 4096, 4096)
        pltpu.sync_copy(tb_u, ou_hbm.at[pl.ds(e0, 4096)])
        pltpu.sync_copy(tb_i, oi_hbm.at[pl.ds(e0, 4096)])

    fire(0, start)

    def body(k, carry):
        process(2 * k, 0)

        @pl.when(2 * k + 1 < count)
        def _():
            process(2 * k + 1, 1)

        return carry

    lax.fori_loop(0, (count + 1) // 2, body, 0)

    @pl.when(wid == 0)
    def _():
        pltpu.sync_copy(utail_hbm, tail_v)
        pltpu.sync_copy(tail_v, ou_hbm.at[pl.ds(TAIL_R0 * 128, 2048)])
        pltpu.sync_copy(itail_hbm, tail_v)
        pltpu.sync_copy(tail_v, oi_hbm.at[pl.ds(TAIL_R0 * 128, 2048)])


def _gmf_body(user_hbm, item_hbm, uw_hbm, iw_hbm, params_hbm, out_hbm,
              uidx_v, iidx_v, udma_v, idma_v, ucol_v, icol_v,
              ub0, ub1, ib0, ib1, params_v, out_v, sem0, sem1):
    wid = lax.axis_index("s") * NC + lax.axis_index("c")
    pltpu.sync_copy(user_hbm.at[wid], uidx_v)
    pltpu.sync_copy(item_hbm.at[wid], iidx_v)
    pltpu.sync_copy(params_hbm, params_v)

    for j in range(NCHUNK):
        for k in range(CHUNK // 16):
            uv = uidx_v[j, pl.ds(k * 16, 16)]
            iv = iidx_v[j, pl.ds(k * 16, 16)]
            udma_v[j, pl.ds(k * 16, 16)] = uv >> 2
            idma_v[j, pl.ds(k * 16, 16)] = iv >> 2
            ucol_v[pl.ds(j * CHUNK + k * 16, 16)] = (uv & 3) * D
            icol_v[pl.ds(j * CHUNK + k * 16, 16)] = (iv & 3) * D

    w_lo = params_v[pl.ds(0, 16)]
    w_hi = params_v[pl.ds(16, 16)]
    bias = params_v[pl.ds(32, 16)][0]
    wd = [w_lo[d] for d in range(16)] + [w_hi[d - 16] for d in range(16, D)]
    lane = jnp.arange(16, dtype=jnp.int32)

    ubufs = [ub0, ub1]
    ibufs = [ib0, ib1]
    sems = [sem0, sem1]

    def fire(c):
        s = sems[c % 2]
        return (pltpu.async_copy(uw_hbm.at[udma_v.at[c]], ubufs[c % 2], s),
                pltpu.async_copy(iw_hbm.at[idma_v.at[c]], ibufs[c % 2], s))

    pending = fire(0)
    for c in range(NCHUNK):
        nxt = fire(c + 1) if c + 1 < NCHUNK else None
        for p in pending:
            p.wait()
        ubuf = ubufs[c % 2]
        ibuf = ibufs[c % 2]

        def body(g, carry):
            rows = g * 16 + lane
            ucol = ucol_v[pl.ds(c * CHUNK + g * 16, 16)]
            icol = icol_v[pl.ds(c * CHUNK + g * 16, 16)]
            acc = jnp.zeros((16,), jnp.float32)
            for d in range(D):
                u = plsc.load_gather(ubuf, [rows, ucol + d])
                i = plsc.load_gather(ibuf, [rows, icol + d])
                acc = acc + (u * i) * wd[d]
            out_v[pl.ds(c * CHUNK + g * 16, 16)] = acc + bias
            return carry

        lax.fori_loop(0, CHUNK // 16, body, 0)
        pending = nxt

    pltpu.sync_copy(out_v, out_hbm.at[wid])


def kernel(user, item, embed_user_weight, embed_item_weight, predict_weight,
           predict_bias):
    uw_t = embed_user_weight.T
    iw_t = embed_item_weight.T
    utail = embed_user_weight[NBLK_FULL * 128:].reshape(2048)
    itail = embed_item_weight[NBLK_FULL * 128:].reshape(2048)
    mesh = plsc.VectorSubcoreMesh(core_axis_name="c", subcore_axis_name="s")
    tr = pl.kernel(
        _transpose_body,
        out_type=(jax.ShapeDtypeStruct((32000000,), jnp.float32),
                  jax.ShapeDtypeStruct((32000000,), jnp.float32)),
        mesh=mesh,
        scratch_types=[
            pltpu.VMEM((D, 128), jnp.float32),
            pltpu.VMEM((D, 128), jnp.float32),
            pltpu.VMEM((D, 128), jnp.float32),
            pltpu.VMEM((D, 128), jnp.float32),
            pltpu.VMEM((4096,), jnp.float32),
            pltpu.VMEM((4096,), jnp.float32),
            pltpu.VMEM((2048,), jnp.float32),
            pltpu.SemaphoreType.DMA,
            pltpu.SemaphoreType.DMA,
        ],
        compiler_params=pltpu.CompilerParams(
            needs_layout_passes=False, use_tc_tiling_on_sc=True),
    )
    uw_f, iw_f = tr(uw_t, iw_t, utail, itail)
    uw_p = uw_f.reshape(250000, 128)
    iw_p = iw_f.reshape(250000, 128)
    user3 = user.reshape(NW, NCHUNK, CHUNK)
    item3 = item.reshape(NW, NCHUNK, CHUNK)
    params = jnp.concatenate([
        predict_weight.reshape(D), predict_bias,
        jnp.zeros((15,), jnp.float32)])
    k = pl.kernel(
        _gmf_body,
        out_type=jax.ShapeDtypeStruct((NW, BPW), jnp.float32),
        mesh=mesh,
        scratch_types=[
            pltpu.VMEM((NCHUNK, CHUNK), jnp.int32),
            pltpu.VMEM((NCHUNK, CHUNK), jnp.int32),
            pltpu.VMEM((NCHUNK, CHUNK), jnp.int32),
            pltpu.VMEM((NCHUNK, CHUNK), jnp.int32),
            pltpu.VMEM((BPW,), jnp.int32),
            pltpu.VMEM((BPW,), jnp.int32),
            pltpu.VMEM((CHUNK, 128), jnp.float32),
            pltpu.VMEM((CHUNK, 128), jnp.float32),
            pltpu.VMEM((CHUNK, 128), jnp.float32),
            pltpu.VMEM((CHUNK, 128), jnp.float32),
            pltpu.VMEM((48,), jnp.float32),
            pltpu.VMEM((BPW,), jnp.float32),
            pltpu.SemaphoreType.DMA,
            pltpu.SemaphoreType.DMA,
        ],
        compiler_params=pltpu.CompilerParams(
            needs_layout_passes=False, use_tc_tiling_on_sc=True),
    )
    out = k(user3, item3, uw_p, iw_p, params)
    return out.reshape(B)


# speedup vs baseline: 1.9733x; 1.5310x over previous
import jax
import jax.numpy as jnp
from jax import lax
from jax.experimental import pallas as pl
from jax.experimental.pallas import tpu as pltpu
from jax.experimental.pallas import tpu_sc as plsc

NC = 2
NS = 16
NW = NC * NS
B = 16384
D = 32
BPW = B // NW
CHUNK = 128
NCHUNK = BPW // CHUNK


def _gmf_body(user_hbm, item_hbm, uw_hbm, iw_hbm, params_hbm, out_hbm,
              uidx_v, iidx_v, urows_v, irows_v, params_v, out_v, sem):
    wid = lax.axis_index("s") * NC + lax.axis_index("c")
    pltpu.sync_copy(user_hbm.at[wid], uidx_v)
    pltpu.sync_copy(item_hbm.at[wid], iidx_v)
    pltpu.sync_copy(params_hbm, params_v)

    copies = []
    for j in range(NCHUNK):
        copies.append(pltpu.async_copy(
            uw_hbm.at[uidx_v.at[j]],
            urows_v.at[pl.ds(j * CHUNK, CHUNK)], sem))
        copies.append(pltpu.async_copy(
            iw_hbm.at[iidx_v.at[j]],
            irows_v.at[pl.ds(j * CHUNK, CHUNK)], sem))
    for c in copies:
        c.wait()

    w_lo = params_v[pl.ds(0, 16)]
    w_hi = params_v[pl.ds(16, 16)]
    bias = params_v[pl.ds(32, 16)][0]
    wd = [w_lo[d] for d in range(16)] + [w_hi[d - 16] for d in range(16, D)]
    lane = jnp.arange(16, dtype=jnp.int32)

    def body(g, carry):
        rows = g * 16 + lane
        acc = jnp.zeros((16,), jnp.float32)
        for d in range(D):
            col = jnp.full((16,), d, jnp.int32)
            u = plsc.load_gather(urows_v, [rows, col])
            i = plsc.load_gather(irows_v, [rows, col])
            acc = acc + (u * i) * wd[d]
        out_v[pl.ds(g * 16, 16)] = acc + bias
        return carry

    lax.fori_loop(0, BPW // 16, body, 0)
    pltpu.sync_copy(out_v, out_hbm.at[wid])


def kernel(user, item, embed_user_weight, embed_item_weight, predict_weight,
           predict_bias):
    user3 = user.reshape(NW, NCHUNK, CHUNK)
    item3 = item.reshape(NW, NCHUNK, CHUNK)
    params = jnp.concatenate([
        predict_weight.reshape(D), predict_bias,
        jnp.zeros((15,), jnp.float32)])
    mesh = plsc.VectorSubcoreMesh(core_axis_name="c", subcore_axis_name="s")
    k = pl.kernel(
        _gmf_body,
        out_type=jax.ShapeDtypeStruct((NW, BPW), jnp.float32),
        mesh=mesh,
        scratch_types=[
            pltpu.VMEM((NCHUNK, CHUNK), jnp.int32),
            pltpu.VMEM((NCHUNK, CHUNK), jnp.int32),
            pltpu.VMEM((BPW, D), jnp.float32),
            pltpu.VMEM((BPW, D), jnp.float32),
            pltpu.VMEM((48,), jnp.float32),
            pltpu.VMEM((BPW,), jnp.float32),
            pltpu.SemaphoreType.DMA,
        ],
        compiler_params=pltpu.CompilerParams(
            needs_layout_passes=False, use_tc_tiling_on_sc=False),
    )
    out = k(user3, item3, embed_user_weight, embed_item_weight, params)
    return out.reshape(B)
